# baseline (speedup 1.0000x reference)
import jax
import jax.numpy as jnp
from jax.experimental import pallas as pl
from jax.experimental.pallas import tpu as pltpu

_PARITIES = ((0, 0), (0, 1), (1, 0), (1, 1))


def _patches(xs, a, b, hs, ws, cin):
    cols = []
    for t in (0, 1):
        for u in (0, 1):
            win = xs[a + t:a + t + hs, b + u:b + u + ws, :]
            cols.append(win.reshape(hs * ws, cin))
    return jnp.concatenate(cols, axis=1)


def _stats_kernel(xp_ref, w_ref, psum_ref, psumsq_ref):
    _, hp, wp, cin = xp_ref.shape
    hs, ws = hp - 2, wp - 2
    xs = xp_ref[0]
    ones = jnp.ones((1, hs * ws), jnp.bfloat16)
    s = jnp.zeros((1, w_ref.shape[2]), jnp.float32)
    s2 = jnp.zeros((1, w_ref.shape[2]), jnp.float32)
    for p, (a, b) in enumerate(_PARITIES):
        patch = _patches(xs, a, b, hs, ws, cin)
        acc = jnp.dot(patch, w_ref[p], preferred_element_type=jnp.float32)
        ab = acc.astype(jnp.bfloat16)
        s = s + jnp.dot(ones, ab, preferred_element_type=jnp.float32)
        s2 = s2 + jnp.dot(ones, ab * ab, preferred_element_type=jnp.float32)
    psum_ref[0] = s
    psumsq_ref[0] = s2


def _apply_kernel(xp_ref, w_ref, shift_ref, o_ref):
    _, hp, wp, cin = xp_ref.shape
    hs, ws = hp - 2, wp - 2
    cout = w_ref.shape[2]
    xs = xp_ref[0]
    zs = []
    for p, (a, b) in enumerate(_PARITIES):
        patch = _patches(xs, a, b, hs, ws, cin)
        acc = jnp.dot(patch, w_ref[p], preferred_element_type=jnp.float32)
        z = jnp.maximum(acc + shift_ref[...], 0.0).astype(jnp.bfloat16)
        zs.append(z.reshape(hs, ws, 1, cout))
    r0 = jnp.concatenate([zs[0], zs[1]], axis=2).reshape(hs, 1, 2 * ws, cout)
    r1 = jnp.concatenate([zs[2], zs[3]], axis=2).reshape(hs, 1, 2 * ws, cout)
    zz = jnp.concatenate([r0, r1], axis=1).reshape(4 * hs * ws, cout)
    o_ref[0] = zz.astype(o_ref.dtype)


def _conv_up_bn_relu(x_nchw, w_oihw, conv_bias, bn_gamma, bn_beta,
                     *, eps=1e-5):
    n, cin, h_in, w_in = x_nchw.shape
    cout = w_oihw.shape[0]
    h, w = 2 * h_in, 2 * w_in
    hw = h * w

    x = jnp.transpose(x_nchw, (0, 2, 3, 1)).astype(jnp.bfloat16)
    xp = jnp.pad(x, ((0, 0), (1, 1), (1, 1), (0, 0)))

    w9 = jnp.transpose(w_oihw, (2, 3, 1, 0)).astype(jnp.float32)
    comb = jnp.array([[[1., 0., 0.], [0., 1., 1.]],
                      [[1., 1., 0.], [0., 0., 1.]]], jnp.float32)
    wf32 = jnp.einsum('atk,bul,klio->abtuio', comb, comb, w9)
    wf32 = wf32.reshape(4, 4 * cin, cout)
    wf = wf32.astype(jnp.bfloat16)

    kb = 4 * cin

    psum, psumsq = pl.pallas_call(
        _stats_kernel,
        out_shape=(
            jax.ShapeDtypeStruct((n, 1, cout), jnp.float32),
            jax.ShapeDtypeStruct((n, 1, cout), jnp.float32),
        ),
        grid=(n,),
        in_specs=[
            pl.BlockSpec((1, h_in + 2, w_in + 2, cin), lambda i: (i, 0, 0, 0)),
            pl.BlockSpec((4, kb, cout), lambda i: (0, 0, 0)),
        ],
        out_specs=(
            pl.BlockSpec((1, 1, cout), lambda i: (i, 0, 0)),
            pl.BlockSpec((1, 1, cout), lambda i: (i, 0, 0)),
        ),
        compiler_params=pltpu.CompilerParams(
            dimension_semantics=("parallel",)),
    )(xp, wf)

    count = jnp.float32(n * hw)
    eacc = jnp.sum(psum[:, 0, :], axis=0) / count
    eacc2 = jnp.sum(psumsq[:, 0, :], axis=0) / count
    bias = conv_bias.astype(jnp.float32)
    mean = eacc + bias
    ex2 = eacc2 + (2.0 * bias) * eacc + bias * bias
    var = jnp.maximum(ex2 - mean * mean, 0.0)
    scale = bn_gamma.astype(jnp.float32) * jax.lax.rsqrt(var + eps)
    shift2 = bn_beta.astype(jnp.float32) - eacc * scale
    wsc = (wf32 * scale[None, None, :]).astype(jnp.bfloat16)

    out = pl.pallas_call(
        _apply_kernel,
        out_shape=jax.ShapeDtypeStruct((n, hw, cout), x_nchw.dtype),
        grid=(n,),
        in_specs=[
            pl.BlockSpec((1, h_in + 2, w_in + 2, cin), lambda i: (i, 0, 0, 0)),
            pl.BlockSpec((4, kb, cout), lambda i: (0, 0, 0)),
            pl.BlockSpec((1, cout), lambda i: (0, 0)),
        ],
        out_specs=pl.BlockSpec((1, hw, cout), lambda i: (i, 0, 0)),
        compiler_params=pltpu.CompilerParams(
            dimension_semantics=("parallel",)),
    )(xp, wsc, shift2.reshape(1, cout))

    return jnp.transpose(out.reshape(n, h, w, cout), (0, 3, 1, 2))


def kernel(x_nchw, w_oihw, conv_bias, bn_gamma, bn_beta):
    return _conv_up_bn_relu(x_nchw, w_oihw, conv_bias, bn_gamma, bn_beta,
                            eps=1e-5)


# speedup vs baseline: 1.4863x; 1.4863x over previous
import jax
import jax.numpy as jnp
from jax.experimental import pallas as pl
from jax.experimental.pallas import tpu as pltpu

_PARITIES = ((0, 0), (0, 1), (1, 0), (1, 1))


def _patches(xs, a, b, hs, ws, cin):
    cols = []
    for t in (0, 1):
        for u in (0, 1):
            win = xs[a + t:a + t + hs, b + u:b + u + ws, :]
            cols.append(win.reshape(hs * ws, cin))
    return jnp.concatenate(cols, axis=1)


def _stats_kernel(xp_ref, w_ref, psum_ref, psumsq_ref):
    _, hp, wp, cin = xp_ref.shape
    hs, ws = hp - 2, wp - 2
    xs = xp_ref[0]
    s = jnp.zeros((1, w_ref.shape[2]), jnp.float32)
    s2 = jnp.zeros((1, w_ref.shape[2]), jnp.float32)
    for p, (a, b) in enumerate(_PARITIES):
        patch = _patches(xs, a, b, hs, ws, cin)
        acc = jnp.dot(patch, w_ref[p], preferred_element_type=jnp.float32)
        s = s + jnp.sum(acc, axis=0, keepdims=True)
        s2 = s2 + jnp.sum(acc * acc, axis=0, keepdims=True)
    psum_ref[0] = s
    psumsq_ref[0] = s2


def _apply_kernel(xp_ref, w_ref, shift_ref, o_ref):
    _, hp, wp, cin = xp_ref.shape
    hs, ws = hp - 2, wp - 2
    cout = w_ref.shape[2]
    xs = xp_ref[0]
    zs = []
    for p, (a, b) in enumerate(_PARITIES):
        patch = _patches(xs, a, b, hs, ws, cin)
        acc = jnp.dot(patch, w_ref[p], preferred_element_type=jnp.float32)
        z = jnp.maximum(acc + shift_ref[...], 0.0)
        zs.append(z.reshape(hs, ws, 1, cout))
    r0 = jnp.concatenate([zs[0], zs[1]], axis=2).reshape(hs, 1, 2 * ws, cout)
    r1 = jnp.concatenate([zs[2], zs[3]], axis=2).reshape(hs, 1, 2 * ws, cout)
    zz = jnp.concatenate([r0, r1], axis=1).reshape(4 * hs * ws, cout)
    o_ref[0] = zz


def _conv_up_bn_relu(x_nchw, w_oihw, conv_bias, bn_gamma, bn_beta,
                     *, eps=1e-5):
    n, cin, h_in, w_in = x_nchw.shape
    cout = w_oihw.shape[0]
    h, w = 2 * h_in, 2 * w_in
    hw = h * w

    x = jnp.transpose(x_nchw, (0, 2, 3, 1)).astype(jnp.bfloat16)
    xp = jnp.pad(x, ((0, 0), (1, 1), (1, 1), (0, 0)))

    w9 = jnp.transpose(w_oihw, (2, 3, 1, 0)).astype(jnp.float32)
    comb = jnp.array([[[1., 0., 0.], [0., 1., 1.]],
                      [[1., 1., 0.], [0., 0., 1.]]], jnp.float32)
    wf32 = jnp.einsum('atk,bul,klio->abtuio', comb, comb, w9)
    wf32 = wf32.reshape(4, 4 * cin, cout)
    wf = wf32.astype(jnp.bfloat16)

    kb = 4 * cin

    psum, psumsq = pl.pallas_call(
        _stats_kernel,
        out_shape=(
            jax.ShapeDtypeStruct((n, 1, cout), jnp.float32),
            jax.ShapeDtypeStruct((n, 1, cout), jnp.float32),
        ),
        grid=(n,),
        in_specs=[
            pl.BlockSpec((1, h_in + 2, w_in + 2, cin), lambda i: (i, 0, 0, 0)),
            pl.BlockSpec((4, kb, cout), lambda i: (0, 0, 0)),
        ],
        out_specs=(
            pl.BlockSpec((1, 1, cout), lambda i: (i, 0, 0)),
            pl.BlockSpec((1, 1, cout), lambda i: (i, 0, 0)),
        ),
        compiler_params=pltpu.CompilerParams(
            dimension_semantics=("parallel",)),
    )(xp, wf)

    count = jnp.float32(n * hw)
    eacc = jnp.sum(psum[:, 0, :], axis=0) / count
    eacc2 = jnp.sum(psumsq[:, 0, :], axis=0) / count
    bias = conv_bias.astype(jnp.float32)
    mean = eacc + bias
    ex2 = eacc2 + (2.0 * bias) * eacc + bias * bias
    var = jnp.maximum(ex2 - mean * mean, 0.0)
    scale = bn_gamma.astype(jnp.float32) * jax.lax.rsqrt(var + eps)
    shift2 = bn_beta.astype(jnp.float32) - eacc * scale
    wsc = (wf32 * scale[None, None, :]).astype(jnp.bfloat16)

    out = pl.pallas_call(
        _apply_kernel,
        out_shape=jax.ShapeDtypeStruct((n, hw, cout), x_nchw.dtype),
        grid=(n,),
        in_specs=[
            pl.BlockSpec((1, h_in + 2, w_in + 2, cin), lambda i: (i, 0, 0, 0)),
            pl.BlockSpec((4, kb, cout), lambda i: (0, 0, 0)),
            pl.BlockSpec((1, cout), lambda i: (0, 0)),
        ],
        out_specs=pl.BlockSpec((1, hw, cout), lambda i: (i, 0, 0)),
        compiler_params=pltpu.CompilerParams(
            dimension_semantics=("parallel",)),
    )(xp, wsc, shift2.reshape(1, cout))

    return jnp.transpose(out.reshape(n, h, w, cout), (0, 3, 1, 2))


def kernel(x_nchw, w_oihw, conv_bias, bn_gamma, bn_beta):
    return _conv_up_bn_relu(x_nchw, w_oihw, conv_bias, bn_gamma, bn_beta,
                            eps=1e-5)


# speedup vs baseline: 2.0227x; 1.3609x over previous
import jax
import jax.numpy as jnp
from jax.experimental import pallas as pl
from jax.experimental.pallas import tpu as pltpu

_PARITIES = ((0, 0), (0, 1), (1, 0), (1, 1))


def _patches(xs, a, b, hs, ws, cin):
    cols = []
    for t in (0, 1):
        for u in (0, 1):
            win = xs[a + t:a + t + hs, b + u:b + u + ws, :]
            cols.append(win.reshape(hs * ws, cin))
    return jnp.concatenate(cols, axis=1)


def _stats_kernel(xp_ref, w_ref, psum_ref, psumsq_ref):
    _, hp, wp, cin = xp_ref.shape
    hs, ws = hp - 2, wp - 2
    xs = xp_ref[0]
    s = jnp.zeros((1, w_ref.shape[2]), jnp.float32)
    s2 = jnp.zeros((1, w_ref.shape[2]), jnp.float32)
    for p, (a, b) in enumerate(_PARITIES):
        patch = _patches(xs, a, b, hs, ws, cin)
        acc = jnp.dot(patch, w_ref[p], preferred_element_type=jnp.float32)
        s = s + jnp.sum(acc, axis=0, keepdims=True)
        s2 = s2 + jnp.sum(acc * acc, axis=0, keepdims=True)
    psum_ref[0] = s
    psumsq_ref[0] = s2


def _apply_kernel(xp_ref, w_ref, shift_ref, o_ref):
    _, hp, wp, cin = xp_ref.shape
    hs, ws = hp - 2, wp - 2
    cout = o_ref.shape[2]
    xs = xp_ref[0]
    rs = []
    for a in (0, 1):
        cols = []
        for t in (0, 1):
            for u in (0, 1, 2):
                win = xs[a + t:a + t + hs, u:u + ws, :]
                cols.append(win.reshape(hs * ws, cin))
        patch = jnp.concatenate(cols, axis=1)
        acc = jnp.dot(patch, w_ref[a], preferred_element_type=jnp.float32)
        z = jnp.maximum(acc + shift_ref[...], 0.0)
        rs.append(z.reshape(hs, 1, 2 * ws, cout))
    zz = jnp.concatenate(rs, axis=1).reshape(4 * hs * ws, cout)
    o_ref[0] = zz


def _conv_up_bn_relu(x_nchw, w_oihw, conv_bias, bn_gamma, bn_beta,
                     *, eps=1e-5):
    n, cin, h_in, w_in = x_nchw.shape
    cout = w_oihw.shape[0]
    h, w = 2 * h_in, 2 * w_in
    hw = h * w

    x = jnp.transpose(x_nchw, (0, 2, 3, 1)).astype(jnp.bfloat16)
    xp = jnp.pad(x, ((0, 0), (1, 1), (1, 1), (0, 0)))

    w9 = jnp.transpose(w_oihw, (2, 3, 1, 0)).astype(jnp.float32)
    comb = jnp.array([[[1., 0., 0.], [0., 1., 1.]],
                      [[1., 1., 0.], [0., 0., 1.]]], jnp.float32)
    wf32 = jnp.einsum('atk,bul,klio->abtuio', comb, comb, w9)
    wf32 = wf32.reshape(4, 4 * cin, cout)
    wf = wf32.astype(jnp.bfloat16)

    kb = 4 * cin

    psum, psumsq = pl.pallas_call(
        _stats_kernel,
        out_shape=(
            jax.ShapeDtypeStruct((n, 1, cout), jnp.float32),
            jax.ShapeDtypeStruct((n, 1, cout), jnp.float32),
        ),
        grid=(n,),
        in_specs=[
            pl.BlockSpec((1, h_in + 2, w_in + 2, cin), lambda i: (i, 0, 0, 0)),
            pl.BlockSpec((4, kb, cout), lambda i: (0, 0, 0)),
        ],
        out_specs=(
            pl.BlockSpec((1, 1, cout), lambda i: (i, 0, 0)),
            pl.BlockSpec((1, 1, cout), lambda i: (i, 0, 0)),
        ),
        compiler_params=pltpu.CompilerParams(
            dimension_semantics=("parallel",)),
    )(xp, wf)

    count = jnp.float32(n * hw)
    eacc = jnp.sum(psum[:, 0, :], axis=0) / count
    eacc2 = jnp.sum(psumsq[:, 0, :], axis=0) / count
    bias = conv_bias.astype(jnp.float32)
    mean = eacc + bias
    ex2 = eacc2 + (2.0 * bias) * eacc + bias * bias
    var = jnp.maximum(ex2 - mean * mean, 0.0)
    scale = bn_gamma.astype(jnp.float32) * jax.lax.rsqrt(var + eps)
    shift2 = bn_beta.astype(jnp.float32) - eacc * scale
    wsc6 = (wf32 * scale[None, None, :]).reshape(2, 2, 2, 2, cin, cout)
    b0 = jnp.pad(wsc6[:, 0], ((0, 0), (0, 0), (0, 1), (0, 0), (0, 0)))
    b1 = jnp.pad(wsc6[:, 1], ((0, 0), (0, 0), (1, 0), (0, 0), (0, 0)))
    w6 = jnp.stack([b0, b1], axis=4)
    w6 = w6.reshape(2, 6 * cin, 2 * cout).astype(jnp.bfloat16)
    shift_b = jnp.concatenate([shift2, shift2]).reshape(1, 2 * cout)

    out = pl.pallas_call(
        _apply_kernel,
        out_shape=jax.ShapeDtypeStruct((n, hw, cout), x_nchw.dtype),
        grid=(n,),
        in_specs=[
            pl.BlockSpec((1, h_in + 2, w_in + 2, cin), lambda i: (i, 0, 0, 0)),
            pl.BlockSpec((2, 6 * cin, 2 * cout), lambda i: (0, 0, 0)),
            pl.BlockSpec((1, 2 * cout), lambda i: (0, 0)),
        ],
        out_specs=pl.BlockSpec((1, hw, cout), lambda i: (i, 0, 0)),
        compiler_params=pltpu.CompilerParams(
            dimension_semantics=("parallel",)),
    )(xp, w6, shift_b)

    return jnp.transpose(out.reshape(n, h, w, cout), (0, 3, 1, 2))


def kernel(x_nchw, w_oihw, conv_bias, bn_gamma, bn_beta):
    return _conv_up_bn_relu(x_nchw, w_oihw, conv_bias, bn_gamma, bn_beta,
                            eps=1e-5)


# speedup vs baseline: 2.4132x; 1.1931x over previous
import jax
import jax.numpy as jnp
from jax.experimental import pallas as pl
from jax.experimental.pallas import tpu as pltpu

_PARITIES = ((0, 0), (0, 1), (1, 0), (1, 1))


def _patches(xs, a, b, hs, ws, cin):
    cols = []
    for t in (0, 1):
        for u in (0, 1):
            win = xs[a + t:a + t + hs, b + u:b + u + ws, :]
            cols.append(win.reshape(hs * ws, cin))
    return jnp.concatenate(cols, axis=1)


def _stats_kernel(xp_ref, w_ref, psum_ref, psumsq_ref):
    nb, hp, wp, cin = xp_ref.shape
    hs, ws = hp - 2, wp - 2
    for k in range(nb):
        xs = xp_ref[k]
        s = jnp.zeros((1, w_ref.shape[2]), jnp.float32)
        s2 = jnp.zeros((1, w_ref.shape[2]), jnp.float32)
        for p, (a, b) in enumerate(_PARITIES):
            patch = _patches(xs, a, b, hs, ws, cin)
            acc = jnp.dot(patch, w_ref[p], preferred_element_type=jnp.float32)
            s = s + jnp.sum(acc, axis=0, keepdims=True)
            s2 = s2 + jnp.sum(acc * acc, axis=0, keepdims=True)
        psum_ref[k] = s
        psumsq_ref[k] = s2


def _apply_kernel(xp_ref, w_ref, shift_ref, o_ref):
    nb, hp, wp, cin = xp_ref.shape
    hs, ws = hp - 2, wp - 2
    cout = o_ref.shape[2]
    for k in range(nb):
        xs = xp_ref[k]
        rs = []
        for a in (0, 1):
            cols = []
            for t in (0, 1):
                for u in (0, 1, 2):
                    win = xs[a + t:a + t + hs, u:u + ws, :]
                    cols.append(win.reshape(hs * ws, cin))
            patch = jnp.concatenate(cols, axis=1)
            acc = jnp.dot(patch, w_ref[a], preferred_element_type=jnp.float32)
            z = jnp.maximum(acc + shift_ref[...], 0.0)
            rs.append(z.reshape(hs, 1, 2 * ws, cout))
        zz = jnp.concatenate(rs, axis=1).reshape(4 * hs * ws, cout)
        o_ref[k] = zz


def _conv_up_bn_relu(x_nchw, w_oihw, conv_bias, bn_gamma, bn_beta,
                     *, eps=1e-5):
    n, cin, h_in, w_in = x_nchw.shape
    cout = w_oihw.shape[0]
    h, w = 2 * h_in, 2 * w_in
    hw = h * w

    x = jnp.transpose(x_nchw, (0, 2, 3, 1)).astype(jnp.bfloat16)
    xp = jnp.pad(x, ((0, 0), (1, 1), (1, 1), (0, 0)))

    w9 = jnp.transpose(w_oihw, (2, 3, 1, 0)).astype(jnp.float32)
    comb = jnp.array([[[1., 0., 0.], [0., 1., 1.]],
                      [[1., 1., 0.], [0., 0., 1.]]], jnp.float32)
    wf32 = jnp.einsum('atk,bul,klio->abtuio', comb, comb, w9)
    wf32 = wf32.reshape(4, 4 * cin, cout)
    wf = wf32.astype(jnp.bfloat16)

    kb = 4 * cin

    psum, psumsq = pl.pallas_call(
        _stats_kernel,
        out_shape=(
            jax.ShapeDtypeStruct((n, 1, cout), jnp.float32),
            jax.ShapeDtypeStruct((n, 1, cout), jnp.float32),
        ),
        grid=(n // 2,),
        in_specs=[
            pl.BlockSpec((2, h_in + 2, w_in + 2, cin), lambda i: (i, 0, 0, 0)),
            pl.BlockSpec((4, kb, cout), lambda i: (0, 0, 0)),
        ],
        out_specs=(
            pl.BlockSpec((2, 1, cout), lambda i: (i, 0, 0)),
            pl.BlockSpec((2, 1, cout), lambda i: (i, 0, 0)),
        ),
        compiler_params=pltpu.CompilerParams(
            dimension_semantics=("parallel",)),
    )(xp, wf)

    count = jnp.float32(n * hw)
    eacc = jnp.sum(psum[:, 0, :], axis=0) / count
    eacc2 = jnp.sum(psumsq[:, 0, :], axis=0) / count
    bias = conv_bias.astype(jnp.float32)
    mean = eacc + bias
    ex2 = eacc2 + (2.0 * bias) * eacc + bias * bias
    var = jnp.maximum(ex2 - mean * mean, 0.0)
    scale = bn_gamma.astype(jnp.float32) * jax.lax.rsqrt(var + eps)
    shift2 = bn_beta.astype(jnp.float32) - eacc * scale
    wsc6 = (wf32 * scale[None, None, :]).reshape(2, 2, 2, 2, cin, cout)
    b0 = jnp.pad(wsc6[:, 0], ((0, 0), (0, 0), (0, 1), (0, 0), (0, 0)))
    b1 = jnp.pad(wsc6[:, 1], ((0, 0), (0, 0), (1, 0), (0, 0), (0, 0)))
    w6 = jnp.stack([b0, b1], axis=4)
    w6 = w6.reshape(2, 6 * cin, 2 * cout).astype(jnp.bfloat16)
    shift_b = jnp.concatenate([shift2, shift2]).reshape(1, 2 * cout)

    out = pl.pallas_call(
        _apply_kernel,
        out_shape=jax.ShapeDtypeStruct((n, hw, cout), x_nchw.dtype),
        grid=(n // 2,),
        in_specs=[
            pl.BlockSpec((2, h_in + 2, w_in + 2, cin), lambda i: (i, 0, 0, 0)),
            pl.BlockSpec((2, 6 * cin, 2 * cout), lambda i: (0, 0, 0)),
            pl.BlockSpec((1, 2 * cout), lambda i: (0, 0)),
        ],
        out_specs=pl.BlockSpec((2, hw, cout), lambda i: (i, 0, 0)),
        compiler_params=pltpu.CompilerParams(
            dimension_semantics=("parallel",)),
    )(xp, w6, shift_b)

    return jnp.transpose(out.reshape(n, h, w, cout), (0, 3, 1, 2))


def kernel(x_nchw, w_oihw, conv_bias, bn_gamma, bn_beta):
    return _conv_up_bn_relu(x_nchw, w_oihw, conv_bias, bn_gamma, bn_beta,
                            eps=1e-5)


# speedup vs baseline: 2.6475x; 1.0971x over previous
import jax
import jax.numpy as jnp
from jax.experimental import pallas as pl
from jax.experimental.pallas import tpu as pltpu

_PARITIES = ((0, 0), (0, 1), (1, 0), (1, 1))


def _patches(xs, a, b, hs, ws, cin):
    cols = []
    for t in (0, 1):
        for u in (0, 1):
            win = xs[a + t:a + t + hs, b + u:b + u + ws, :]
            cols.append(win.reshape(hs * ws, cin))
    return jnp.concatenate(cols, axis=1)


def _stats_kernel(xp_ref, w_ref, psum_ref, psumsq_ref):
    nb, hp, wp, cin = xp_ref.shape
    hs, ws = hp - 2, wp - 2
    for k in range(nb):
        xs = xp_ref[k]
        s = jnp.zeros((1, w_ref.shape[2]), jnp.float32)
        s2 = jnp.zeros((1, w_ref.shape[2]), jnp.float32)
        for p, (a, b) in enumerate(_PARITIES):
            patch = _patches(xs, a, b, hs, ws, cin)
            acc = jnp.dot(patch, w_ref[p], preferred_element_type=jnp.float32)
            s = s + jnp.sum(acc, axis=0, keepdims=True)
            s2 = s2 + jnp.sum(acc * acc, axis=0, keepdims=True)
        psum_ref[k] = s
        psumsq_ref[k] = s2


def _apply_kernel(xp_ref, w_ref, shift_ref, o_ref):
    nb, hp, wp, cin = xp_ref.shape
    hs, ws = hp - 2, wp - 2
    cout = o_ref.shape[2]
    for k in range(nb):
        xs = xp_ref[k]
        rs = []
        for a in (0, 1):
            cols = []
            for t in (0, 1):
                for u in (0, 1, 2):
                    win = xs[a + t:a + t + hs, u:u + ws, :]
                    cols.append(win.reshape(hs * ws, cin))
            patch = jnp.concatenate(cols, axis=1)
            acc = jnp.dot(patch, w_ref[a], preferred_element_type=jnp.float32)
            z = jnp.maximum(acc + shift_ref[...], 0.0)
            rs.append(z.reshape(hs, 1, 2 * ws, cout))
        zz = jnp.concatenate(rs, axis=1).reshape(4 * hs * ws, cout)
        o_ref[k] = zz


def _conv_up_bn_relu(x_nchw, w_oihw, conv_bias, bn_gamma, bn_beta,
                     *, eps=1e-5):
    n, cin, h_in, w_in = x_nchw.shape
    cout = w_oihw.shape[0]
    h, w = 2 * h_in, 2 * w_in
    hw = h * w

    x = jnp.transpose(x_nchw, (0, 2, 3, 1)).astype(jnp.bfloat16)
    xp = jnp.pad(x, ((0, 0), (1, 1), (1, 1), (0, 0)))

    w9 = jnp.transpose(w_oihw, (2, 3, 1, 0)).astype(jnp.float32)
    comb = jnp.array([[[1., 0., 0.], [0., 1., 1.]],
                      [[1., 1., 0.], [0., 0., 1.]]], jnp.float32)
    wf32 = jnp.einsum('atk,bul,klio->abtuio', comb, comb, w9)
    wf32 = wf32.reshape(4, 4 * cin, cout)
    wf = wf32.astype(jnp.bfloat16)

    kb = 4 * cin

    psum, psumsq = pl.pallas_call(
        _stats_kernel,
        out_shape=(
            jax.ShapeDtypeStruct((n, 1, cout), jnp.float32),
            jax.ShapeDtypeStruct((n, 1, cout), jnp.float32),
        ),
        grid=(n // 4,),
        in_specs=[
            pl.BlockSpec((4, h_in + 2, w_in + 2, cin), lambda i: (i, 0, 0, 0)),
            pl.BlockSpec((4, kb, cout), lambda i: (0, 0, 0)),
        ],
        out_specs=(
            pl.BlockSpec((4, 1, cout), lambda i: (i, 0, 0)),
            pl.BlockSpec((4, 1, cout), lambda i: (i, 0, 0)),
        ),
        compiler_params=pltpu.CompilerParams(
            dimension_semantics=("parallel",)),
    )(xp, wf)

    count = jnp.float32(n * hw)
    eacc = jnp.sum(psum[:, 0, :], axis=0) / count
    eacc2 = jnp.sum(psumsq[:, 0, :], axis=0) / count
    bias = conv_bias.astype(jnp.float32)
    mean = eacc + bias
    ex2 = eacc2 + (2.0 * bias) * eacc + bias * bias
    var = jnp.maximum(ex2 - mean * mean, 0.0)
    scale = bn_gamma.astype(jnp.float32) * jax.lax.rsqrt(var + eps)
    shift2 = bn_beta.astype(jnp.float32) - eacc * scale
    wsc6 = (wf32 * scale[None, None, :]).reshape(2, 2, 2, 2, cin, cout)
    b0 = jnp.pad(wsc6[:, 0], ((0, 0), (0, 0), (0, 1), (0, 0), (0, 0)))
    b1 = jnp.pad(wsc6[:, 1], ((0, 0), (0, 0), (1, 0), (0, 0), (0, 0)))
    w6 = jnp.stack([b0, b1], axis=4)
    w6 = w6.reshape(2, 6 * cin, 2 * cout).astype(jnp.bfloat16)
    shift_b = jnp.concatenate([shift2, shift2]).reshape(1, 2 * cout)

    out = pl.pallas_call(
        _apply_kernel,
        out_shape=jax.ShapeDtypeStruct((n, hw, cout), x_nchw.dtype),
        grid=(n // 4,),
        in_specs=[
            pl.BlockSpec((4, h_in + 2, w_in + 2, cin), lambda i: (i, 0, 0, 0)),
            pl.BlockSpec((2, 6 * cin, 2 * cout), lambda i: (0, 0, 0)),
            pl.BlockSpec((1, 2 * cout), lambda i: (0, 0)),
        ],
        out_specs=pl.BlockSpec((4, hw, cout), lambda i: (i, 0, 0)),
        compiler_params=pltpu.CompilerParams(
            dimension_semantics=("parallel",)),
    )(xp, w6, shift_b)

    return jnp.transpose(out.reshape(n, h, w, cout), (0, 3, 1, 2))


def kernel(x_nchw, w_oihw, conv_bias, bn_gamma, bn_beta):
    return _conv_up_bn_relu(x_nchw, w_oihw, conv_bias, bn_gamma, bn_beta,
                            eps=1e-5)


# speedup vs baseline: 2.6840x; 1.0138x over previous
import jax
import jax.numpy as jnp
from jax.experimental import pallas as pl
from jax.experimental.pallas import tpu as pltpu

_PARITIES = ((0, 0), (0, 1), (1, 0), (1, 1))


def _patches(xs, a, b, hs, ws, cin):
    cols = []
    for t in (0, 1):
        for u in (0, 1):
            win = xs[a + t:a + t + hs, b + u:b + u + ws, :]
            cols.append(win.reshape(hs * ws, cin))
    return jnp.concatenate(cols, axis=1)


def _stats_kernel(xp_ref, w_ref, psum_ref, psumsq_ref):
    nb, hp, wp, cin = xp_ref.shape
    hs, ws = hp - 2, wp - 2
    for k in range(nb):
        xs = xp_ref[k]
        s = jnp.zeros((1, w_ref.shape[2]), jnp.float32)
        s2 = jnp.zeros((1, w_ref.shape[2]), jnp.float32)
        for p, (a, b) in enumerate(_PARITIES):
            patch = _patches(xs, a, b, hs, ws, cin)
            acc = jnp.dot(patch, w_ref[p], preferred_element_type=jnp.float32)
            s = s + jnp.sum(acc, axis=0, keepdims=True)
            s2 = s2 + jnp.sum(acc * acc, axis=0, keepdims=True)
        psum_ref[k] = s
        psumsq_ref[k] = s2


def _apply_kernel(xp_ref, w_ref, shift_ref, o_ref):
    nb, hp, wp, cin = xp_ref.shape
    hs, ws = hp - 2, wp - 2
    cout = o_ref.shape[2]
    for k in range(nb):
        xs = xp_ref[k]
        rs = []
        for a in (0, 1):
            cols = []
            for t in (0, 1):
                for u in (0, 1, 2):
                    win = xs[a + t:a + t + hs, u:u + ws, :]
                    cols.append(win.reshape(hs * ws, cin))
            patch = jnp.concatenate(cols, axis=1)
            acc = jnp.dot(patch, w_ref[a], preferred_element_type=jnp.float32)
            z = jnp.maximum(acc + shift_ref[...], 0.0)
            rs.append(z.reshape(hs, 1, 2 * ws, cout))
        zz = jnp.concatenate(rs, axis=1).reshape(4 * hs * ws, cout)
        o_ref[k] = zz


def _conv_up_bn_relu(x_nchw, w_oihw, conv_bias, bn_gamma, bn_beta,
                     *, eps=1e-5):
    n, cin, h_in, w_in = x_nchw.shape
    cout = w_oihw.shape[0]
    h, w = 2 * h_in, 2 * w_in
    hw = h * w

    x = jnp.transpose(x_nchw, (0, 2, 3, 1)).astype(jnp.bfloat16)
    xp = jnp.pad(x, ((0, 0), (1, 1), (1, 1), (0, 0)))

    w9 = jnp.transpose(w_oihw, (2, 3, 1, 0)).astype(jnp.float32)
    comb = jnp.array([[[1., 0., 0.], [0., 1., 1.]],
                      [[1., 1., 0.], [0., 0., 1.]]], jnp.float32)
    wf32 = jnp.einsum('atk,bul,klio->abtuio', comb, comb, w9)
    wf32 = wf32.reshape(4, 4 * cin, cout)
    wf = wf32.astype(jnp.bfloat16)

    kb = 4 * cin

    psum, psumsq = pl.pallas_call(
        _stats_kernel,
        out_shape=(
            jax.ShapeDtypeStruct((n, 1, cout), jnp.float32),
            jax.ShapeDtypeStruct((n, 1, cout), jnp.float32),
        ),
        grid=(n // 8,),
        in_specs=[
            pl.BlockSpec((8, h_in + 2, w_in + 2, cin), lambda i: (i, 0, 0, 0)),
            pl.BlockSpec((4, kb, cout), lambda i: (0, 0, 0)),
        ],
        out_specs=(
            pl.BlockSpec((8, 1, cout), lambda i: (i, 0, 0)),
            pl.BlockSpec((8, 1, cout), lambda i: (i, 0, 0)),
        ),
        compiler_params=pltpu.CompilerParams(
            dimension_semantics=("parallel",)),
    )(xp, wf)

    count = jnp.float32(n * hw)
    eacc = jnp.sum(psum[:, 0, :], axis=0) / count
    eacc2 = jnp.sum(psumsq[:, 0, :], axis=0) / count
    bias = conv_bias.astype(jnp.float32)
    mean = eacc + bias
    ex2 = eacc2 + (2.0 * bias) * eacc + bias * bias
    var = jnp.maximum(ex2 - mean * mean, 0.0)
    scale = bn_gamma.astype(jnp.float32) * jax.lax.rsqrt(var + eps)
    shift2 = bn_beta.astype(jnp.float32) - eacc * scale
    wsc6 = (wf32 * scale[None, None, :]).reshape(2, 2, 2, 2, cin, cout)
    b0 = jnp.pad(wsc6[:, 0], ((0, 0), (0, 0), (0, 1), (0, 0), (0, 0)))
    b1 = jnp.pad(wsc6[:, 1], ((0, 0), (0, 0), (1, 0), (0, 0), (0, 0)))
    w6 = jnp.stack([b0, b1], axis=4)
    w6 = w6.reshape(2, 6 * cin, 2 * cout).astype(jnp.bfloat16)
    shift_b = jnp.concatenate([shift2, shift2]).reshape(1, 2 * cout)

    out = pl.pallas_call(
        _apply_kernel,
        out_shape=jax.ShapeDtypeStruct((n, hw, cout), x_nchw.dtype),
        grid=(n // 8,),
        in_specs=[
            pl.BlockSpec((8, h_in + 2, w_in + 2, cin), lambda i: (i, 0, 0, 0)),
            pl.BlockSpec((2, 6 * cin, 2 * cout), lambda i: (0, 0, 0)),
            pl.BlockSpec((1, 2 * cout), lambda i: (0, 0)),
        ],
        out_specs=pl.BlockSpec((8, hw, cout), lambda i: (i, 0, 0)),
        compiler_params=pltpu.CompilerParams(
            dimension_semantics=("parallel",)),
    )(xp, w6, shift_b)

    return jnp.transpose(out.reshape(n, h, w, cout), (0, 3, 1, 2))


def kernel(x_nchw, w_oihw, conv_bias, bn_gamma, bn_beta):
    return _conv_up_bn_relu(x_nchw, w_oihw, conv_bias, bn_gamma, bn_beta,
                            eps=1e-5)
